# Initial kernel scaffold; baseline (speedup 1.0000x reference)
import jax
import jax.numpy as jnp
from jax.experimental import pallas as pl


def kernel(x, grad, Wr, br, W1, b1, W2, b2):
    raise NotImplementedError("write your pallas kernel here")


# speedup vs baseline: 1.8266x; 1.8266x over previous
import functools

import jax
import jax.numpy as jnp
from jax import lax
from jax.experimental import pallas as pl
from jax.experimental.pallas import tpu as pltpu


def _router_body(x_ref, g_ref, wrx_ref, wrg_ref, br_ref, w_ref):
    x = x_ref[...]
    logits = jnp.dot(x, wrx_ref[...], preferred_element_type=jnp.float32)
    logits = logits + g_ref[...] * wrg_ref[...] + br_ref[...]
    m = jnp.max(logits, axis=1, keepdims=True)
    ex = jnp.exp(logits - m)
    p = ex / jnp.sum(ex, axis=1, keepdims=True)
    m1 = jnp.max(p, axis=1, keepdims=True)
    masked = jnp.where(p == m1, -jnp.inf, p)
    m2 = jnp.max(masked, axis=1, keepdims=True)
    w_ref[...] = jnp.where(p >= m2, p, 0.0)


def _ffn_body(x_ref, w_ref, w1_ref, b1_ref, w2_ref, b2_ref, out_ref):
    e = pl.program_id(1)
    h = pl.program_id(2)
    num_e = pl.num_programs(1)
    x = x_ref[...]
    hpre = jnp.dot(x, w1_ref[0], preferred_element_type=jnp.float32)
    hpre = hpre + b1_ref[0]
    act = (0.5 * hpre * (1.0 + lax.erf(hpre * 0.7071067811865476))
           ).astype(jnp.bfloat16)
    y = jnp.dot(act, w2_ref[0], preferred_element_type=jnp.float32)
    eidx = lax.broadcasted_iota(jnp.int32, (1, num_e), 1)
    wcol = jnp.sum(w_ref[...] * (eidx == e).astype(jnp.float32),
                   axis=1, keepdims=True)
    sel_b2 = (h == 0).astype(jnp.float32)
    contrib = wcol * (y + sel_b2 * b2_ref[0])
    first = jnp.logical_and(e == 0, h == 0)

    @pl.when(first)
    def _():
        out_ref[...] = contrib

    @pl.when(jnp.logical_not(first))
    def _():
        out_ref[...] = out_ref[...] + contrib


def _router_weights(x, grad, Wr, br, tn):
    n, d = x.shape
    e = Wr.shape[1]
    wrx = Wr[:d]
    wrg = Wr[d:].reshape(1, e)
    br2 = br.reshape(1, e)
    return pl.pallas_call(
        _router_body,
        grid=(n // tn,),
        in_specs=[
            pl.BlockSpec((tn, d), lambda i: (i, 0)),
            pl.BlockSpec((tn, 1), lambda i: (i, 0)),
            pl.BlockSpec((d, e), lambda i: (0, 0)),
            pl.BlockSpec((1, e), lambda i: (0, 0)),
            pl.BlockSpec((1, e), lambda i: (0, 0)),
        ],
        out_specs=pl.BlockSpec((tn, e), lambda i: (i, 0)),
        out_shape=jax.ShapeDtypeStruct((n, e), jnp.float32),
    )(x, grad, wrx, wrg, br2)


def kernel(x, grad, Wr, br, W1, b1, W2, b2):
    n, d = x.shape
    e, _, hd = W1.shape
    tn = min(512, n)
    th = min(512, hd)

    w = _router_weights(x, grad, Wr, br, tn)

    xb = x.astype(jnp.bfloat16)
    w1b = W1.astype(jnp.bfloat16)
    w2b = W2.astype(jnp.bfloat16)
    b1f = b1.reshape(e, 1, hd).astype(jnp.float32)
    b2f = b2.reshape(e, 1, d).astype(jnp.float32)

    out = pl.pallas_call(
        _ffn_body,
        grid=(n // tn, e, hd // th),
        in_specs=[
            pl.BlockSpec((tn, d), lambda i, j, k: (i, 0)),
            pl.BlockSpec((tn, e), lambda i, j, k: (i, 0)),
            pl.BlockSpec((1, d, th), lambda i, j, k: (j, 0, k)),
            pl.BlockSpec((1, 1, th), lambda i, j, k: (j, 0, k)),
            pl.BlockSpec((1, th, d), lambda i, j, k: (j, k, 0)),
            pl.BlockSpec((1, 1, d), lambda i, j, k: (j, 0, 0)),
        ],
        out_specs=pl.BlockSpec((tn, d), lambda i, j, k: (i, 0)),
        out_shape=jax.ShapeDtypeStruct((n, d), jnp.float32),
        compiler_params=pltpu.CompilerParams(
            dimension_semantics=("parallel", "arbitrary", "arbitrary"),
        ),
    )(xb, w, w1b, b1f, w2b, b2f)
    return out


# speedup vs baseline: 2.0472x; 1.1208x over previous
import functools

import jax
import jax.numpy as jnp
from jax import lax
from jax.experimental import pallas as pl
from jax.experimental.pallas import tpu as pltpu


def _router_body(x_ref, g_ref, wrx_ref, wrg_ref, br_ref, w_ref):
    x = x_ref[...]
    logits = jnp.dot(x, wrx_ref[...], preferred_element_type=jnp.float32)
    logits = logits + g_ref[...] * wrg_ref[...] + br_ref[...]
    m = jnp.max(logits, axis=1, keepdims=True)
    ex = jnp.exp(logits - m)
    p = ex / jnp.sum(ex, axis=1, keepdims=True)
    m1 = jnp.max(p, axis=1, keepdims=True)
    masked = jnp.where(p == m1, -jnp.inf, p)
    m2 = jnp.max(masked, axis=1, keepdims=True)
    w_ref[...] = jnp.where(p >= m2, p, 0.0)


def _ffn_body(x_ref, w_ref, w1_ref, b1_ref, w2_ref, b2_ref, out_ref):
    e = pl.program_id(1)
    h = pl.program_id(2)
    num_e = pl.num_programs(1)
    x = x_ref[...]
    hpre = jnp.dot(x, w1_ref[0], preferred_element_type=jnp.float32)
    hpre = hpre + b1_ref[0]
    act = (0.5 * hpre * (1.0 + lax.erf(hpre * 0.7071067811865476))
           ).astype(jnp.bfloat16)
    y = jnp.dot(act, w2_ref[0], preferred_element_type=jnp.float32)
    eidx = lax.broadcasted_iota(jnp.int32, (1, num_e), 1)
    wcol = jnp.sum(w_ref[...] * (eidx == e).astype(jnp.float32),
                   axis=1, keepdims=True)
    sel_b2 = (h == 0).astype(jnp.float32)
    contrib = wcol * (y + sel_b2 * b2_ref[0])
    first = jnp.logical_and(e == 0, h == 0)

    @pl.when(first)
    def _():
        out_ref[...] = contrib

    @pl.when(jnp.logical_not(first))
    def _():
        out_ref[...] = out_ref[...] + contrib


def _router_weights(x, grad, Wr, br, tn):
    n, d = x.shape
    e = Wr.shape[1]
    wrx = Wr[:d]
    wrg = Wr[d:].reshape(1, e)
    br2 = br.reshape(1, e)
    return pl.pallas_call(
        _router_body,
        grid=(n // tn,),
        in_specs=[
            pl.BlockSpec((tn, d), lambda i: (i, 0)),
            pl.BlockSpec((tn, 1), lambda i: (i, 0)),
            pl.BlockSpec((d, e), lambda i: (0, 0)),
            pl.BlockSpec((1, e), lambda i: (0, 0)),
            pl.BlockSpec((1, e), lambda i: (0, 0)),
        ],
        out_specs=pl.BlockSpec((tn, e), lambda i: (i, 0)),
        out_shape=jax.ShapeDtypeStruct((n, e), jnp.float32),
    )(x, grad, wrx, wrg, br2)


def kernel(x, grad, Wr, br, W1, b1, W2, b2):
    n, d = x.shape
    e, _, hd = W1.shape
    tn = min(1024, n)
    th = min(512, hd)

    w = _router_weights(x, grad, Wr, br, tn)

    xb = x.astype(jnp.bfloat16)
    w1b = W1.astype(jnp.bfloat16)
    w2b = W2.astype(jnp.bfloat16)
    b1f = b1.reshape(e, 1, hd).astype(jnp.float32)
    b2f = b2.reshape(e, 1, d).astype(jnp.float32)

    out = pl.pallas_call(
        _ffn_body,
        grid=(n // tn, e, hd // th),
        in_specs=[
            pl.BlockSpec((tn, d), lambda i, j, k: (i, 0)),
            pl.BlockSpec((tn, e), lambda i, j, k: (i, 0)),
            pl.BlockSpec((1, d, th), lambda i, j, k: (j, 0, k)),
            pl.BlockSpec((1, 1, th), lambda i, j, k: (j, 0, k)),
            pl.BlockSpec((1, th, d), lambda i, j, k: (j, k, 0)),
            pl.BlockSpec((1, 1, d), lambda i, j, k: (j, 0, 0)),
        ],
        out_specs=pl.BlockSpec((tn, d), lambda i, j, k: (i, 0)),
        out_shape=jax.ShapeDtypeStruct((n, d), jnp.float32),
        compiler_params=pltpu.CompilerParams(
            dimension_semantics=("parallel", "arbitrary", "arbitrary"),
        ),
    )(xb, w, w1b, b1f, w2b, b2f)
    return out


# speedup vs baseline: 2.1874x; 1.0685x over previous
import functools

import jax
import jax.numpy as jnp
from jax import lax
from jax.experimental import pallas as pl
from jax.experimental.pallas import tpu as pltpu
from jax.experimental.pallas import tpu_sc as plsc

_NC = 2
_NS = 16
_NW = _NC * _NS


def _router_body(x_ref, g_ref, wrx_ref, wrg_ref, br_ref,
                 probs_ref, idx_ref, rank_ref, counts_ref, carry):
    pid = pl.program_id(0)

    @pl.when(pid == 0)
    def _():
        carry[...] = jnp.zeros_like(carry)

    x = x_ref[...]
    logits = jnp.dot(x, wrx_ref[...], preferred_element_type=jnp.float32)
    logits = logits + g_ref[...] * wrg_ref[...] + br_ref[...]
    m = jnp.max(logits, axis=1, keepdims=True)
    ex = jnp.exp(logits - m)
    p = ex / jnp.sum(ex, axis=1, keepdims=True)

    tn, ne = p.shape
    lane = lax.broadcasted_iota(jnp.int32, (tn, ne), 1)
    m1 = jnp.max(p, axis=1, keepdims=True)
    i1 = jnp.min(jnp.where(p == m1, lane, ne), axis=1, keepdims=True)
    sel1 = lane == i1
    pm = jnp.where(sel1, -jnp.inf, p)
    m2 = jnp.max(pm, axis=1, keepdims=True)
    i2 = jnp.min(jnp.where(pm == m2, lane, ne), axis=1, keepdims=True)
    sel2 = lane == i2

    probs_ref[...] = jnp.concatenate([m1, m2], axis=1)
    idx_ref[...] = jnp.concatenate([i1, i2], axis=1)

    oh = sel1.astype(jnp.float32) + sel2.astype(jnp.float32)
    row = lax.broadcasted_iota(jnp.int32, (tn, tn), 0)
    col = lax.broadcasted_iota(jnp.int32, (tn, tn), 1)
    tri = (row > col).astype(jnp.float32)
    cum = jnp.dot(tri, oh, preferred_element_type=jnp.float32) + carry[...]
    r1 = jnp.sum(jnp.where(sel1, cum, 0.0), axis=1, keepdims=True)
    r2 = jnp.sum(jnp.where(sel2, cum, 0.0), axis=1, keepdims=True)
    rank_ref[...] = jnp.concatenate([r1, r2], axis=1).astype(jnp.int32)

    carry[...] = carry[...] + jnp.sum(oh, axis=0, keepdims=True)
    counts_ref[...] = carry[...].astype(jnp.int32)


def _router(x, grad, Wr, br, tn):
    n, d = x.shape
    e = Wr.shape[1]
    wrx = Wr[:d]
    wrg = Wr[d:].reshape(1, e)
    br2 = br.reshape(1, e)
    return pl.pallas_call(
        _router_body,
        grid=(n // tn,),
        in_specs=[
            pl.BlockSpec((tn, d), lambda i: (i, 0)),
            pl.BlockSpec((tn, 1), lambda i: (i, 0)),
            pl.BlockSpec((d, e), lambda i: (0, 0)),
            pl.BlockSpec((1, e), lambda i: (0, 0)),
            pl.BlockSpec((1, e), lambda i: (0, 0)),
        ],
        out_specs=[
            pl.BlockSpec((tn, 2), lambda i: (i, 0)),
            pl.BlockSpec((tn, 2), lambda i: (i, 0)),
            pl.BlockSpec((tn, 2), lambda i: (i, 0)),
            pl.BlockSpec((1, e), lambda i: (0, 0)),
        ],
        out_shape=[
            jax.ShapeDtypeStruct((n, 2), jnp.float32),
            jax.ShapeDtypeStruct((n, 2), jnp.int32),
            jax.ShapeDtypeStruct((n, 2), jnp.int32),
            jax.ShapeDtypeStruct((1, e), jnp.int32),
        ],
        scratch_shapes=[pltpu.VMEM((1, e), jnp.float32)],
    )(x, grad, wrx, wrg, br2)


def _make_sc_gather(n, d, b, dtype):
    bpw = b // _NW
    ch = min(32, bpw)
    nchunk = bpw // ch
    mesh = plsc.VectorSubcoreMesh(core_axis_name="c", subcore_axis_name="s")

    @functools.partial(
        pl.kernel,
        out_type=jax.ShapeDtypeStruct((b, d), dtype),
        mesh=mesh,
        scratch_types=[
            pltpu.VMEM((ch,), jnp.int32),
            pltpu.VMEM((ch, d), dtype),
            pltpu.SemaphoreType.DMA,
        ],
    )
    def gather_k(x_hbm, src_hbm, xg_hbm, idx_v, buf, sem):
        wid = lax.axis_index("s") * _NC + lax.axis_index("c")
        base = wid * bpw
        for i in range(nchunk):
            off = base + i * ch
            pltpu.sync_copy(src_hbm.at[pl.ds(off, ch)], idx_v)
            pltpu.async_copy(x_hbm.at[idx_v], buf, sem).wait()
            pltpu.sync_copy(buf, xg_hbm.at[pl.ds(off, ch)])

    return gather_k


def _make_sc_combine(n, d, b):
    npw = n // _NW
    ch = min(32, npw)
    nchunk = npw // ch
    nvec = d // 16
    mesh = plsc.VectorSubcoreMesh(core_axis_name="c", subcore_axis_name="s")

    @functools.partial(
        pl.kernel,
        out_type=jax.ShapeDtypeStruct((n, d), jnp.float32),
        mesh=mesh,
        scratch_types=[
            pltpu.VMEM((ch,), jnp.int32),
            pltpu.VMEM((ch,), jnp.int32),
            pltpu.VMEM((ch, d), jnp.float32),
            pltpu.VMEM((ch, d), jnp.float32),
            pltpu.SemaphoreType.DMA,
            pltpu.SemaphoreType.DMA,
        ],
    )
    def combine_k(y_hbm, pos0_hbm, pos1_hbm, out_hbm,
                  i0_v, i1_v, b0, b1, sem0, sem1):
        wid = lax.axis_index("s") * _NC + lax.axis_index("c")
        base = wid * npw
        for c in range(nchunk):
            off = base + c * ch
            pltpu.sync_copy(pos0_hbm.at[pl.ds(off, ch)], i0_v)
            pltpu.sync_copy(pos1_hbm.at[pl.ds(off, ch)], i1_v)
            cp0 = pltpu.async_copy(y_hbm.at[i0_v], b0, sem0)
            cp1 = pltpu.async_copy(y_hbm.at[i1_v], b1, sem1)
            cp0.wait()
            cp1.wait()

            def body(r, _):
                def inner(j, _):
                    sl = pl.ds(j * 16, 16)
                    b0[r, sl] = b0[r, sl] + b1[r, sl]
                    return 0
                return lax.fori_loop(0, nvec, inner, 0)

            lax.fori_loop(0, ch, body, 0)
            pltpu.sync_copy(b0, out_hbm.at[pl.ds(off, ch)])

    return combine_k


def _ffn_body(te_ref, xg_ref, ps_ref, w1_ref, b1_ref, w2_ref, b2_ref, y_ref):
    h = pl.program_id(1)
    nh = pl.num_programs(1)
    x = xg_ref[...].astype(jnp.bfloat16)
    hp = jnp.dot(x, w1_ref[0], preferred_element_type=jnp.float32)
    hp = hp + b1_ref[0]
    act = (0.5 * hp * (1.0 + lax.erf(hp * 0.7071067811865476))
           ).astype(jnp.bfloat16)
    y = jnp.dot(act, w2_ref[0], preferred_element_type=jnp.float32)

    @pl.when(h == 0)
    def _():
        y_ref[...] = y + b2_ref[0]

    @pl.when(h > 0)
    def _():
        y_ref[...] = y_ref[...] + y

    @pl.when(h == nh - 1)
    def _():
        y_ref[...] = y_ref[...] * ps_ref[...]


def _grouped_ffn(te, xg, pscale, W1b, b1f, W2b, b2f, t, th):
    b, d = xg.shape
    e, _, hd = W1b.shape
    grid_spec = pltpu.PrefetchScalarGridSpec(
        num_scalar_prefetch=1,
        grid=(b // t, hd // th),
        in_specs=[
            pl.BlockSpec((t, d), lambda i, j, te_r: (i, 0)),
            pl.BlockSpec((t, 1), lambda i, j, te_r: (i, 0)),
            pl.BlockSpec((1, d, th), lambda i, j, te_r: (te_r[i], 0, j)),
            pl.BlockSpec((1, 1, th), lambda i, j, te_r: (te_r[i], 0, j)),
            pl.BlockSpec((1, th, d), lambda i, j, te_r: (te_r[i], j, 0)),
            pl.BlockSpec((1, 1, d), lambda i, j, te_r: (te_r[i], 0, 0)),
        ],
        out_specs=pl.BlockSpec((t, d), lambda i, j, te_r: (i, 0)),
    )
    return pl.pallas_call(
        _ffn_body,
        grid_spec=grid_spec,
        out_shape=jax.ShapeDtypeStruct((b, d), jnp.float32),
        compiler_params=pltpu.CompilerParams(
            dimension_semantics=("arbitrary", "arbitrary"),
        ),
    )(te, xg, pscale, W1b, b1f, W2b, b2f)


def kernel(x, grad, Wr, br, W1, b1, W2, b2):
    n, d = x.shape
    e, _, hd = W1.shape
    t = min(512, max(n // 8, 8))
    th = min(512, hd)
    nk = 2 * n
    b = nk + e * t

    probs, idx, rank, counts2d = _router(x, grad, Wr, br, min(1024, n))

    counts = counts2d[0]
    aligned = ((counts + t - 1) // t) * t
    ends = jnp.cumsum(aligned)
    seg_off = ends - aligned
    pos = jnp.take(seg_off, idx, axis=0) + rank
    posf = pos.reshape(-1)
    tok = jnp.repeat(jnp.arange(n, dtype=jnp.int32), 2)
    src = jnp.zeros((b,), jnp.int32).at[posf].set(tok)
    pscale = jnp.zeros((b,), jnp.float32).at[posf].set(
        probs.reshape(-1)).reshape(b, 1)
    tile_starts = jnp.arange(b // t, dtype=jnp.int32) * t
    te = jnp.sum((tile_starts[:, None] >= ends[None, :]).astype(jnp.int32),
                 axis=1)
    te = jnp.minimum(te, e - 1)

    xg = _make_sc_gather(n, d, b, jnp.float32)(x, src)

    w1b = W1.astype(jnp.bfloat16)
    w2b = W2.astype(jnp.bfloat16)
    b1f = b1.reshape(e, 1, hd).astype(jnp.float32)
    b2f = b2.reshape(e, 1, d).astype(jnp.float32)
    y = _grouped_ffn(te, xg, pscale, w1b, b1f, w2b, b2f, t, th)

    out = _make_sc_combine(n, d, b)(y, pos[:, 0], pos[:, 1])
    return out


# speedup vs baseline: 2.2489x; 1.0281x over previous
import functools

import jax
import jax.numpy as jnp
from jax import lax
from jax.experimental import pallas as pl
from jax.experimental.pallas import tpu as pltpu
from jax.experimental.pallas import tpu_sc as plsc

_NC = 2
_NS = 16
_NW = _NC * _NS


def _router_body(x_ref, g_ref, wrx_ref, wrg_ref, br_ref,
                 probs_ref, idx_ref, rank_ref, counts_ref, carry):
    pid = pl.program_id(0)

    @pl.when(pid == 0)
    def _():
        carry[...] = jnp.zeros_like(carry)

    x = x_ref[...]
    logits = jnp.dot(x, wrx_ref[...], preferred_element_type=jnp.float32)
    logits = logits + g_ref[...] * wrg_ref[...] + br_ref[...]
    m = jnp.max(logits, axis=1, keepdims=True)
    ex = jnp.exp(logits - m)
    p = ex / jnp.sum(ex, axis=1, keepdims=True)

    tn, ne = p.shape
    lane = lax.broadcasted_iota(jnp.int32, (tn, ne), 1)
    m1 = jnp.max(p, axis=1, keepdims=True)
    i1 = jnp.min(jnp.where(p == m1, lane, ne), axis=1, keepdims=True)
    sel1 = lane == i1
    pm = jnp.where(sel1, -jnp.inf, p)
    m2 = jnp.max(pm, axis=1, keepdims=True)
    i2 = jnp.min(jnp.where(pm == m2, lane, ne), axis=1, keepdims=True)
    sel2 = lane == i2

    probs_ref[...] = jnp.concatenate([m1, m2], axis=1)
    idx_ref[...] = jnp.concatenate([i1, i2], axis=1)

    oh = sel1.astype(jnp.float32) + sel2.astype(jnp.float32)
    row = lax.broadcasted_iota(jnp.int32, (tn, tn), 0)
    col = lax.broadcasted_iota(jnp.int32, (tn, tn), 1)
    tri = (row > col).astype(jnp.float32)
    cum = jnp.dot(tri, oh, preferred_element_type=jnp.float32) + carry[...]
    r1 = jnp.sum(jnp.where(sel1, cum, 0.0), axis=1, keepdims=True)
    r2 = jnp.sum(jnp.where(sel2, cum, 0.0), axis=1, keepdims=True)
    rank_ref[...] = jnp.concatenate([r1, r2], axis=1).astype(jnp.int32)

    carry[...] = carry[...] + jnp.sum(oh, axis=0, keepdims=True)
    counts_ref[...] = carry[...].astype(jnp.int32)


def _router(x, grad, Wr, br, tn):
    n, d = x.shape
    e = Wr.shape[1]
    wrx = Wr[:d]
    wrg = Wr[d:].reshape(1, e)
    br2 = br.reshape(1, e)
    return pl.pallas_call(
        _router_body,
        grid=(n // tn,),
        in_specs=[
            pl.BlockSpec((tn, d), lambda i: (i, 0)),
            pl.BlockSpec((tn, 1), lambda i: (i, 0)),
            pl.BlockSpec((d, e), lambda i: (0, 0)),
            pl.BlockSpec((1, e), lambda i: (0, 0)),
            pl.BlockSpec((1, e), lambda i: (0, 0)),
        ],
        out_specs=[
            pl.BlockSpec((tn, 2), lambda i: (i, 0)),
            pl.BlockSpec((tn, 2), lambda i: (i, 0)),
            pl.BlockSpec((tn, 2), lambda i: (i, 0)),
            pl.BlockSpec((1, e), lambda i: (0, 0)),
        ],
        out_shape=[
            jax.ShapeDtypeStruct((n, 2), jnp.float32),
            jax.ShapeDtypeStruct((n, 2), jnp.int32),
            jax.ShapeDtypeStruct((n, 2), jnp.int32),
            jax.ShapeDtypeStruct((1, e), jnp.int32),
        ],
        scratch_shapes=[pltpu.VMEM((1, e), jnp.float32)],
    )(x, grad, wrx, wrg, br2)


def _make_sc_gather(n, d, b, dtype):
    bpw = b // _NW
    ch = min(32, bpw)
    nchunk = bpw // ch
    mesh = plsc.VectorSubcoreMesh(core_axis_name="c", subcore_axis_name="s")

    @functools.partial(
        pl.kernel,
        out_type=jax.ShapeDtypeStruct((b, d), dtype),
        mesh=mesh,
        scratch_types=[
            pltpu.VMEM((ch,), jnp.int32),
            pltpu.VMEM((ch, d), dtype),
            pltpu.SemaphoreType.DMA,
        ],
    )
    def gather_k(x_hbm, src_hbm, xg_hbm, idx_v, buf, sem):
        wid = lax.axis_index("s") * _NC + lax.axis_index("c")
        base = wid * bpw
        for i in range(nchunk):
            off = base + i * ch
            pltpu.sync_copy(src_hbm.at[pl.ds(off, ch)], idx_v)
            pltpu.async_copy(x_hbm.at[idx_v], buf, sem).wait()
            pltpu.sync_copy(buf, xg_hbm.at[pl.ds(off, ch)])

    return gather_k


def _make_sc_combine(n, d, b):
    npw = n // _NW
    ch = min(32, npw)
    nchunk = npw // ch
    nvec = d // 16
    mesh = plsc.VectorSubcoreMesh(core_axis_name="c", subcore_axis_name="s")

    @functools.partial(
        pl.kernel,
        out_type=jax.ShapeDtypeStruct((n, d), jnp.float32),
        mesh=mesh,
        scratch_types=[
            pltpu.VMEM((ch,), jnp.int32),
            pltpu.VMEM((ch,), jnp.int32),
            pltpu.VMEM((ch, d), jnp.float32),
            pltpu.VMEM((ch, d), jnp.float32),
            pltpu.SemaphoreType.DMA,
            pltpu.SemaphoreType.DMA,
        ],
    )
    def combine_k(y_hbm, pos0_hbm, pos1_hbm, out_hbm,
                  i0_v, i1_v, b0, b1, sem0, sem1):
        wid = lax.axis_index("s") * _NC + lax.axis_index("c")
        base = wid * npw
        for c in range(nchunk):
            off = base + c * ch
            pltpu.sync_copy(pos0_hbm.at[pl.ds(off, ch)], i0_v)
            pltpu.sync_copy(pos1_hbm.at[pl.ds(off, ch)], i1_v)
            cp0 = pltpu.async_copy(y_hbm.at[i0_v], b0, sem0)
            cp1 = pltpu.async_copy(y_hbm.at[i1_v], b1, sem1)
            cp0.wait()
            cp1.wait()

            def body(r, _):
                for j in range(nvec):
                    sl = pl.ds(j * 16, 16)
                    b0[r, sl] = b0[r, sl] + b1[r, sl]
                return 0

            lax.fori_loop(0, ch, body, 0)
            pltpu.sync_copy(b0, out_hbm.at[pl.ds(off, ch)])

    return combine_k


def _ffn_body(te_ref, xg_ref, ps_ref, w1_ref, b1_ref, w2_ref, b2_ref, y_ref):
    h = pl.program_id(1)
    nh = pl.num_programs(1)
    x = xg_ref[...].astype(jnp.bfloat16)
    hp = jnp.dot(x, w1_ref[0], preferred_element_type=jnp.float32)
    hp = hp + b1_ref[0]
    act = (0.5 * hp * (1.0 + lax.erf(hp * 0.7071067811865476))
           ).astype(jnp.bfloat16)
    y = jnp.dot(act, w2_ref[0], preferred_element_type=jnp.float32)

    @pl.when(h == 0)
    def _():
        y_ref[...] = y + b2_ref[0]

    @pl.when(h > 0)
    def _():
        y_ref[...] = y_ref[...] + y

    @pl.when(h == nh - 1)
    def _():
        y_ref[...] = y_ref[...] * ps_ref[...]


def _grouped_ffn(te, xg, pscale, W1b, b1f, W2b, b2f, t, th):
    b, d = xg.shape
    e, _, hd = W1b.shape
    grid_spec = pltpu.PrefetchScalarGridSpec(
        num_scalar_prefetch=1,
        grid=(b // t, hd // th),
        in_specs=[
            pl.BlockSpec((t, d), lambda i, j, te_r: (i, 0)),
            pl.BlockSpec((t, 1), lambda i, j, te_r: (i, 0)),
            pl.BlockSpec((1, d, th), lambda i, j, te_r: (te_r[i], 0, j)),
            pl.BlockSpec((1, 1, th), lambda i, j, te_r: (te_r[i], 0, j)),
            pl.BlockSpec((1, th, d), lambda i, j, te_r: (te_r[i], j, 0)),
            pl.BlockSpec((1, 1, d), lambda i, j, te_r: (te_r[i], 0, 0)),
        ],
        out_specs=pl.BlockSpec((t, d), lambda i, j, te_r: (i, 0)),
    )
    return pl.pallas_call(
        _ffn_body,
        grid_spec=grid_spec,
        out_shape=jax.ShapeDtypeStruct((b, d), jnp.float32),
        compiler_params=pltpu.CompilerParams(
            dimension_semantics=("arbitrary", "arbitrary"),
        ),
    )(te, xg, pscale, W1b, b1f, W2b, b2f)


def kernel(x, grad, Wr, br, W1, b1, W2, b2):
    n, d = x.shape
    e, _, hd = W1.shape
    t = min(512, max(n // 8, 8))
    th = min(512, hd)
    nk = 2 * n
    b = nk + e * t

    probs, idx, rank, counts2d = _router(x, grad, Wr, br, min(1024, n))

    counts = counts2d[0]
    aligned = ((counts + t - 1) // t) * t
    ends = jnp.cumsum(aligned)
    seg_off = ends - aligned
    pos = jnp.take(seg_off, idx, axis=0) + rank
    posf = pos.reshape(-1)
    tok = jnp.repeat(jnp.arange(n, dtype=jnp.int32), 2)
    src = jnp.zeros((b,), jnp.int32).at[posf].set(tok)
    pscale = jnp.zeros((b,), jnp.float32).at[posf].set(
        probs.reshape(-1)).reshape(b, 1)
    tile_starts = jnp.arange(b // t, dtype=jnp.int32) * t
    te = jnp.sum((tile_starts[:, None] >= ends[None, :]).astype(jnp.int32),
                 axis=1)
    te = jnp.minimum(te, e - 1)

    xg = _make_sc_gather(n, d, b, jnp.float32)(x, src)

    w1b = W1.astype(jnp.bfloat16)
    w2b = W2.astype(jnp.bfloat16)
    b1f = b1.reshape(e, 1, hd).astype(jnp.float32)
    b2f = b2.reshape(e, 1, d).astype(jnp.float32)
    y = _grouped_ffn(te, xg, pscale, w1b, b1f, w2b, b2f, t, th)

    out = _make_sc_combine(n, d, b)(y, pos[:, 0], pos[:, 1])
    return out
